# baseline (speedup 1.0000x reference)
import functools

import jax
import jax.numpy as jnp
from jax import lax
from jax.experimental import pallas as pl
from jax.experimental.pallas import tpu as pltpu
from jax.experimental.pallas import tpu_sc as plsc

N = 8192
DIN = 384
DH = 128
K = 8
RB = 256
CB = 2048
NRB = N // RB
NCB = N // CB
SCALE = DH ** -0.5
NEG = float("-inf")

_PC = pl.pallas_call


def _lrelu(t):
    return jnp.where(t >= 0, t, 0.01 * t)


def _s1_body(x_ref, w_ref, b_ref, x1_ref, csum_ref):
    i = pl.program_id(0)
    z = jnp.dot(x_ref[...], w_ref[...], preferred_element_type=jnp.float32)
    x1 = _lrelu(z + b_ref[...])
    x1_ref[...] = x1

    @pl.when(i == 0)
    def _():
        csum_ref[...] = jnp.zeros_like(csum_ref)

    csum_ref[...] += jnp.sum(x1, axis=0, keepdims=True)


def _stage1(x, W1, b1):
    return _PC(
        _s1_body,
        grid=(NRB,),
        in_specs=[
            pl.BlockSpec((RB, DIN), lambda i: (i, 0)),
            pl.BlockSpec((DIN, DH), lambda i: (0, 0)),
            pl.BlockSpec((1, DH), lambda i: (0, 0)),
        ],
        out_specs=[
            pl.BlockSpec((RB, DH), lambda i: (i, 0)),
            pl.BlockSpec((1, DH), lambda i: (0, 0)),
        ],
        out_shape=[
            jax.ShapeDtypeStruct((N, DH), jnp.float32),
            jax.ShapeDtypeStruct((1, DH), jnp.float32),
        ],
    )(x, W1, b1)


def _s2_body(x1_ref, csum_ref, wh_ref, bh_ref, wt_ref, bt_ref, eh_ref, et_ref):
    x2 = (x1_ref[...] + csum_ref[...] * (1.0 / N)) * 0.5
    eh_ref[...] = jnp.dot(x2, wh_ref[...], preferred_element_type=jnp.float32) + bh_ref[...]
    et_ref[...] = jnp.dot(x2, wt_ref[...], preferred_element_type=jnp.float32) + bt_ref[...]


def _stage2(x1, csum, Wh, bh, Wt, bt):
    return _PC(
        _s2_body,
        grid=(NRB,),
        in_specs=[
            pl.BlockSpec((RB, DH), lambda i: (i, 0)),
            pl.BlockSpec((1, DH), lambda i: (0, 0)),
            pl.BlockSpec((DH, DH), lambda i: (0, 0)),
            pl.BlockSpec((1, DH), lambda i: (0, 0)),
            pl.BlockSpec((DH, DH), lambda i: (0, 0)),
            pl.BlockSpec((1, DH), lambda i: (0, 0)),
        ],
        out_specs=[
            pl.BlockSpec((RB, DH), lambda i: (i, 0)),
            pl.BlockSpec((RB, DH), lambda i: (i, 0)),
        ],
        out_shape=[
            jax.ShapeDtypeStruct((N, DH), jnp.float32),
            jax.ShapeDtypeStruct((N, DH), jnp.float32),
        ],
    )(x1, csum, Wh, bh, Wt, bt)


def _s3_body(eh_ref, et_ref, tw_ref, ti_ref):
    ehs = eh_ref[...] * SCALE
    iota_s = lax.broadcasted_iota(jnp.int32, (CB, RB), 0)
    iota_k = lax.broadcasted_iota(jnp.int32, (K, RB), 0)

    def col_step(c, carry):
        vals, ids = carry
        et_t = et_ref[pl.ds(c * CB, CB), :]
        logits = lax.dot_general(
            et_t, ehs, (((1,), (1,)), ((), ())),
            preferred_element_type=jnp.float32)

        def round_cond(st):
            r, work, m, vals, ids = st
            return jnp.logical_and(r < K, jnp.any(m > vals[K - 1:K, :]))

        def round_body(st):
            r, work, m, vals, ids = st
            cand = work == m
            pos = jnp.min(jnp.where(cand, iota_s, CB), axis=0, keepdims=True)
            work = jnp.where(iota_s == pos, NEG, work)
            cval = m
            cidx = pos + c * CB
            cnt = jnp.sum((vals >= cval).astype(jnp.int32), axis=0, keepdims=True)
            sv = jnp.concatenate(
                [jnp.full((1, RB), NEG, jnp.float32), vals[: K - 1, :]], axis=0)
            si = jnp.concatenate(
                [jnp.zeros((1, RB), jnp.int32), ids[: K - 1, :]], axis=0)
            newv = jnp.where(iota_k < cnt, vals,
                             jnp.where(iota_k == cnt, cval, sv))
            newi = jnp.where(iota_k < cnt, ids,
                             jnp.where(iota_k == cnt, cidx, si))
            m2 = jnp.max(work, axis=0, keepdims=True)
            return (r + 1, work, m2, newv, newi)

        m0 = jnp.max(logits, axis=0, keepdims=True)
        _, _, _, vals, ids = lax.while_loop(
            round_cond, round_body, (0, logits, m0, vals, ids))
        return (vals, ids)

    vals0 = jnp.full((K, RB), NEG, jnp.float32)
    ids0 = jnp.zeros((K, RB), jnp.int32)
    vals, ids = lax.fori_loop(0, NCB, col_step, (vals0, ids0))
    tw_ref[...] = vals
    ti_ref[...] = ids


def _stage3(eh, et):
    return _PC(
        _s3_body,
        grid=(NRB,),
        in_specs=[
            pl.BlockSpec((RB, DH), lambda i: (i, 0)),
            pl.BlockSpec((N, DH), lambda i: (0, 0)),
        ],
        out_specs=[
            pl.BlockSpec((K, RB), lambda i: (0, i)),
            pl.BlockSpec((K, RB), lambda i: (0, i)),
        ],
        out_shape=[
            jax.ShapeDtypeStruct((K, N), jnp.float32),
            jax.ShapeDtypeStruct((K, N), jnp.int32),
        ],
    )(eh, et)


_SC_CHUNK = 128


def _sc_gather(table, idx_flat):
    info = plsc.get_sparse_core_info()
    nw = info.num_cores * info.num_subcores
    b_per_w = (N * K) // nw
    n_chunks = b_per_w // _SC_CHUNK

    @functools.partial(
        pl.kernel,
        out_type=jax.ShapeDtypeStruct((N * K, DH), jnp.float32),
        mesh=plsc.VectorSubcoreMesh(core_axis_name="c", subcore_axis_name="s"),
        scratch_types=[
            pltpu.VMEM((_SC_CHUNK,), jnp.int32),
            pltpu.VMEM((_SC_CHUNK, DH), jnp.float32),
            pltpu.SemaphoreType.DMA,
        ],
    )
    def gather_kernel(table_hbm, idx_hbm, out_hbm, idx_v, rows_v, sem):
        wid = lax.axis_index("s") * info.num_cores + lax.axis_index("c")
        base = wid * b_per_w

        def body(ci, carry):
            off = base + ci * _SC_CHUNK
            pltpu.sync_copy(idx_hbm.at[pl.ds(off, _SC_CHUNK)], idx_v)
            pltpu.async_copy(table_hbm.at[idx_v], rows_v, sem).wait()
            pltpu.sync_copy(rows_v, out_hbm.at[pl.ds(off, _SC_CHUNK)])
            return carry

        lax.fori_loop(0, n_chunks, body, 0)

    return gather_kernel(table, idx_flat)


def _s5_body(eh_ref, nb_ref, tw_ref, l1_ref, l1b_ref, l2_ref, l2b_ref,
             a1_ref, a1b_ref, a2t_ref, a2b_ref, h_ref, gl_ref):
    eh = eh_ref[...]
    nb = nb_ref[...].reshape(RB, K, DH)
    tw = tw_ref[...]

    mw = jnp.max(tw, axis=1, keepdims=True)
    ew = jnp.exp(tw - mw)
    p = ew / jnp.sum(ew, axis=1, keepdims=True)

    eh3 = eh[:, None, :]
    ehr = p[:, :, None] * nb + (1.0 - p)[:, :, None] * eh3
    gate = jnp.tanh(eh3 + ehr)
    kw = jnp.sum(nb, axis=2) * jnp.sum(gate, axis=2)
    mk = jnp.max(kw, axis=1, keepdims=True)
    ek = jnp.exp(kw - mk)
    kp = ek / jnp.sum(ek, axis=1, keepdims=True)
    e_nh = jnp.sum(kp[:, :, None] * nb, axis=1)

    s_emb = _lrelu(jnp.dot(eh + e_nh, l1_ref[...],
                           preferred_element_type=jnp.float32) + l1b_ref[...])
    b_emb = _lrelu(jnp.dot(eh * e_nh, l2_ref[...],
                           preferred_element_type=jnp.float32) + l2b_ref[...])
    h = s_emb + b_emb
    h_ref[...] = h

    a = _lrelu(jnp.dot(h, a1_ref[...],
                       preferred_element_type=jnp.float32) + a1b_ref[...])
    gl = lax.dot_general(a2t_ref[...], a, (((1,), (1,)), ((), ())),
                         preferred_element_type=jnp.float32)
    gl_ref[...] = gl + a2b_ref[...]


def _stage5(eh, nb, tw, L1, l1b, L2, l2b, A1, a1b, A2t, a2b):
    return _PC(
        _s5_body,
        grid=(NRB,),
        in_specs=[
            pl.BlockSpec((RB, DH), lambda i: (i, 0)),
            pl.BlockSpec((RB * K, DH), lambda i: (i, 0)),
            pl.BlockSpec((RB, K), lambda i: (i, 0)),
            pl.BlockSpec((DH, DH), lambda i: (0, 0)),
            pl.BlockSpec((1, DH), lambda i: (0, 0)),
            pl.BlockSpec((DH, DH), lambda i: (0, 0)),
            pl.BlockSpec((1, DH), lambda i: (0, 0)),
            pl.BlockSpec((DH, DH // 2), lambda i: (0, 0)),
            pl.BlockSpec((1, DH // 2), lambda i: (0, 0)),
            pl.BlockSpec((1, DH // 2), lambda i: (0, 0)),
            pl.BlockSpec((1, 1), lambda i: (0, 0)),
        ],
        out_specs=[
            pl.BlockSpec((RB, DH), lambda i: (i, 0)),
            pl.BlockSpec((1, RB), lambda i: (0, i)),
        ],
        out_shape=[
            jax.ShapeDtypeStruct((N, DH), jnp.float32),
            jax.ShapeDtypeStruct((1, N), jnp.float32),
        ],
    )(eh, nb, tw, L1, l1b, L2, l2b, A1, a1b, A2t, a2b)


def _s6_body(gl_ref, h_ref, lng_ref, lnb_ref, wc_ref, bc_ref,
             logits_ref, prob_ref, yhat_ref):
    gl = gl_ref[...]
    m = jnp.max(gl)
    e = jnp.exp(gl - m)
    alpha = e / jnp.sum(e)
    pooled = jnp.dot(alpha, h_ref[...],
                     preferred_element_type=jnp.float32)
    mu = jnp.mean(pooled, axis=1, keepdims=True)
    var = jnp.mean((pooled - mu) ** 2, axis=1, keepdims=True)
    normed = (pooled - mu) / jnp.sqrt(var + 1e-5) * lng_ref[...] + lnb_ref[...]
    logits = jnp.dot(normed, wc_ref[...],
                     preferred_element_type=jnp.float32) + bc_ref[...]
    logits_ref[...] = logits
    ml = jnp.max(logits, axis=1, keepdims=True)
    el = jnp.exp(logits - ml)
    prob_ref[...] = el / jnp.sum(el, axis=1, keepdims=True)
    iota2 = lax.broadcasted_iota(jnp.int32, (1, 2), 1)
    yhat_ref[...] = jnp.min(jnp.where(logits == ml, iota2, 2),
                            axis=1, keepdims=True)


def _stage6(gl, h, ln_g, ln_b, Wc, bc):
    return _PC(
        _s6_body,
        grid=(1,),
        in_specs=[
            pl.BlockSpec((1, N), lambda i: (0, 0)),
            pl.BlockSpec((N, DH), lambda i: (0, 0)),
            pl.BlockSpec((1, DH), lambda i: (0, 0)),
            pl.BlockSpec((1, DH), lambda i: (0, 0)),
            pl.BlockSpec((DH, 2), lambda i: (0, 0)),
            pl.BlockSpec((1, 2), lambda i: (0, 0)),
        ],
        out_specs=[
            pl.BlockSpec((1, 2), lambda i: (0, 0)),
            pl.BlockSpec((1, 2), lambda i: (0, 0)),
            pl.BlockSpec((1, 1), lambda i: (0, 0)),
        ],
        out_shape=[
            jax.ShapeDtypeStruct((1, 2), jnp.float32),
            jax.ShapeDtypeStruct((1, 2), jnp.float32),
            jax.ShapeDtypeStruct((1, 1), jnp.int32),
        ],
    )(gl, h, ln_g, ln_b, Wc, bc)


def kernel(x, W1, b1, Wh, bh, Wt, bt, L1, l1b, L2, l2b,
           A1, a1b, A2, a2b, ln_g, ln_b, Wc, bc):
    b1r = b1.reshape(1, DH)
    bhr = bh.reshape(1, DH)
    btr = bt.reshape(1, DH)
    l1br = l1b.reshape(1, DH)
    l2br = l2b.reshape(1, DH)
    a1br = a1b.reshape(1, DH // 2)
    a2t = A2.reshape(1, DH // 2)
    a2br = a2b.reshape(1, 1)
    lngr = ln_g.reshape(1, DH)
    lnbr = ln_b.reshape(1, DH)
    bcr = bc.reshape(1, 2)

    x1, csum = _stage1(x, W1, b1r)
    eh, et = _stage2(x1, csum, Wh, bhr, Wt, btr)
    tw_t, ti_t = _stage3(eh, et)
    tw = tw_t.T
    ti = ti_t.T
    nb = _sc_gather(et, ti.reshape(N * K))
    h, gl = _stage5(eh, nb, tw, L1, l1br, L2, l2br, A1, a1br, a2t, a2br)
    logits, prob, yhat = _stage6(gl, h, lngr, lnbr, Wc, bcr)
    return (logits, prob, yhat)


# speedup vs baseline: 1.1343x; 1.1343x over previous
import functools

import jax
import jax.numpy as jnp
from jax import lax
from jax.experimental import pallas as pl
from jax.experimental.pallas import tpu as pltpu
from jax.experimental.pallas import tpu_sc as plsc

N = 8192
DIN = 384
DH = 128
K = 8
RB = 256
CB = 1024
RB3 = 512
NRB3 = N // RB3
NRB = N // RB
NCB = N // CB
SCALE = DH ** -0.5
NEG = float("-inf")

_PC = pl.pallas_call


def _lrelu(t):
    return jnp.where(t >= 0, t, 0.01 * t)


def _s1_body(x_ref, w_ref, b_ref, x1_ref, csum_ref):
    i = pl.program_id(0)
    z = jnp.dot(x_ref[...], w_ref[...], preferred_element_type=jnp.float32)
    x1 = _lrelu(z + b_ref[...])
    x1_ref[...] = x1

    @pl.when(i == 0)
    def _():
        csum_ref[...] = jnp.zeros_like(csum_ref)

    csum_ref[...] += jnp.sum(x1, axis=0, keepdims=True)


def _stage1(x, W1, b1):
    return _PC(
        _s1_body,
        grid=(NRB,),
        in_specs=[
            pl.BlockSpec((RB, DIN), lambda i: (i, 0)),
            pl.BlockSpec((DIN, DH), lambda i: (0, 0)),
            pl.BlockSpec((1, DH), lambda i: (0, 0)),
        ],
        out_specs=[
            pl.BlockSpec((RB, DH), lambda i: (i, 0)),
            pl.BlockSpec((1, DH), lambda i: (0, 0)),
        ],
        out_shape=[
            jax.ShapeDtypeStruct((N, DH), jnp.float32),
            jax.ShapeDtypeStruct((1, DH), jnp.float32),
        ],
    )(x, W1, b1)


def _s2_body(x1_ref, csum_ref, wh_ref, bh_ref, wt_ref, bt_ref, eh_ref, et_ref):
    x2 = (x1_ref[...] + csum_ref[...] * (1.0 / N)) * 0.5
    eh_ref[...] = jnp.dot(x2, wh_ref[...], preferred_element_type=jnp.float32) + bh_ref[...]
    et_ref[...] = jnp.dot(x2, wt_ref[...], preferred_element_type=jnp.float32) + bt_ref[...]


def _stage2(x1, csum, Wh, bh, Wt, bt):
    return _PC(
        _s2_body,
        grid=(NRB,),
        in_specs=[
            pl.BlockSpec((RB, DH), lambda i: (i, 0)),
            pl.BlockSpec((1, DH), lambda i: (0, 0)),
            pl.BlockSpec((DH, DH), lambda i: (0, 0)),
            pl.BlockSpec((1, DH), lambda i: (0, 0)),
            pl.BlockSpec((DH, DH), lambda i: (0, 0)),
            pl.BlockSpec((1, DH), lambda i: (0, 0)),
        ],
        out_specs=[
            pl.BlockSpec((RB, DH), lambda i: (i, 0)),
            pl.BlockSpec((RB, DH), lambda i: (i, 0)),
        ],
        out_shape=[
            jax.ShapeDtypeStruct((N, DH), jnp.float32),
            jax.ShapeDtypeStruct((N, DH), jnp.float32),
        ],
    )(x1, csum, Wh, bh, Wt, bt)


def _s3_body(eh_ref, et_ref, tw_ref, ti_ref):
    ehs = eh_ref[...] * SCALE
    iota_s = lax.broadcasted_iota(jnp.int32, (CB, RB3), 0)
    iota_k = lax.broadcasted_iota(jnp.int32, (K, RB3), 0)

    def col_step(c, carry):
        vals, ids = carry
        et_t = et_ref[pl.ds(c * CB, CB), :]
        logits = lax.dot_general(
            et_t, ehs, (((1,), (1,)), ((), ())),
            preferred_element_type=jnp.float32)

        def round_cond(st):
            r, work, m, vals, ids = st
            return jnp.logical_and(r < K, jnp.any(m > vals[K - 1:K, :]))

        def round_body(st):
            r, work, m, vals, ids = st
            cand = work == m
            pos = jnp.min(jnp.where(cand, iota_s, CB), axis=0, keepdims=True)
            work = jnp.where(iota_s == pos, NEG, work)
            cval = m
            cidx = pos + c * CB
            cnt = jnp.sum((vals >= cval).astype(jnp.int32), axis=0, keepdims=True)
            sv = jnp.concatenate(
                [jnp.full((1, RB3), NEG, jnp.float32), vals[: K - 1, :]], axis=0)
            si = jnp.concatenate(
                [jnp.zeros((1, RB3), jnp.int32), ids[: K - 1, :]], axis=0)
            newv = jnp.where(iota_k < cnt, vals,
                             jnp.where(iota_k == cnt, cval, sv))
            newi = jnp.where(iota_k < cnt, ids,
                             jnp.where(iota_k == cnt, cidx, si))
            m2 = jnp.max(work, axis=0, keepdims=True)
            return (r + 1, work, m2, newv, newi)

        m0 = jnp.max(logits, axis=0, keepdims=True)
        _, _, _, vals, ids = lax.while_loop(
            round_cond, round_body, (0, logits, m0, vals, ids))
        return (vals, ids)

    vals0 = jnp.full((K, RB3), NEG, jnp.float32)
    ids0 = jnp.zeros((K, RB3), jnp.int32)
    vals, ids = lax.fori_loop(0, NCB, col_step, (vals0, ids0))
    tw_ref[...] = vals
    ti_ref[...] = ids


def _stage3(eh, et):
    return _PC(
        _s3_body,
        grid=(NRB3,),
        in_specs=[
            pl.BlockSpec((RB3, DH), lambda i: (i, 0)),
            pl.BlockSpec((N, DH), lambda i: (0, 0)),
        ],
        out_specs=[
            pl.BlockSpec((K, RB3), lambda i: (0, i)),
            pl.BlockSpec((K, RB3), lambda i: (0, i)),
        ],
        out_shape=[
            jax.ShapeDtypeStruct((K, N), jnp.float32),
            jax.ShapeDtypeStruct((K, N), jnp.int32),
        ],
    )(eh, et)


_SC_CHUNK = 128


def _sc_gather(table, idx_flat):
    info = plsc.get_sparse_core_info()
    nw = info.num_cores * info.num_subcores
    b_per_w = (N * K) // nw
    n_chunks = b_per_w // _SC_CHUNK

    @functools.partial(
        pl.kernel,
        out_type=jax.ShapeDtypeStruct((N * K, DH), jnp.float32),
        mesh=plsc.VectorSubcoreMesh(core_axis_name="c", subcore_axis_name="s"),
        scratch_types=[
            pltpu.VMEM((_SC_CHUNK,), jnp.int32),
            pltpu.VMEM((_SC_CHUNK, DH), jnp.float32),
            pltpu.SemaphoreType.DMA,
        ],
    )
    def gather_kernel(table_hbm, idx_hbm, out_hbm, idx_v, rows_v, sem):
        wid = lax.axis_index("s") * info.num_cores + lax.axis_index("c")
        base = wid * b_per_w

        def body(ci, carry):
            off = base + ci * _SC_CHUNK
            pltpu.sync_copy(idx_hbm.at[pl.ds(off, _SC_CHUNK)], idx_v)
            pltpu.async_copy(table_hbm.at[idx_v], rows_v, sem).wait()
            pltpu.sync_copy(rows_v, out_hbm.at[pl.ds(off, _SC_CHUNK)])
            return carry

        lax.fori_loop(0, n_chunks, body, 0)

    return gather_kernel(table, idx_flat)


def _s5_body(eh_ref, nb_ref, tw_ref, l1_ref, l1b_ref, l2_ref, l2b_ref,
             a1_ref, a1b_ref, a2t_ref, a2b_ref, h_ref, gl_ref):
    eh = eh_ref[...]
    nb = nb_ref[...].reshape(RB, K, DH)
    tw = tw_ref[...]

    mw = jnp.max(tw, axis=1, keepdims=True)
    ew = jnp.exp(tw - mw)
    p = ew / jnp.sum(ew, axis=1, keepdims=True)

    eh3 = eh[:, None, :]
    ehr = p[:, :, None] * nb + (1.0 - p)[:, :, None] * eh3
    gate = jnp.tanh(eh3 + ehr)
    kw = jnp.sum(nb, axis=2) * jnp.sum(gate, axis=2)
    mk = jnp.max(kw, axis=1, keepdims=True)
    ek = jnp.exp(kw - mk)
    kp = ek / jnp.sum(ek, axis=1, keepdims=True)
    e_nh = jnp.sum(kp[:, :, None] * nb, axis=1)

    s_emb = _lrelu(jnp.dot(eh + e_nh, l1_ref[...],
                           preferred_element_type=jnp.float32) + l1b_ref[...])
    b_emb = _lrelu(jnp.dot(eh * e_nh, l2_ref[...],
                           preferred_element_type=jnp.float32) + l2b_ref[...])
    h = s_emb + b_emb
    h_ref[...] = h

    a = _lrelu(jnp.dot(h, a1_ref[...],
                       preferred_element_type=jnp.float32) + a1b_ref[...])
    gl = lax.dot_general(a2t_ref[...], a, (((1,), (1,)), ((), ())),
                         preferred_element_type=jnp.float32)
    gl_ref[...] = gl + a2b_ref[...]


def _stage5(eh, nb, tw, L1, l1b, L2, l2b, A1, a1b, A2t, a2b):
    return _PC(
        _s5_body,
        grid=(NRB,),
        in_specs=[
            pl.BlockSpec((RB, DH), lambda i: (i, 0)),
            pl.BlockSpec((RB * K, DH), lambda i: (i, 0)),
            pl.BlockSpec((RB, K), lambda i: (i, 0)),
            pl.BlockSpec((DH, DH), lambda i: (0, 0)),
            pl.BlockSpec((1, DH), lambda i: (0, 0)),
            pl.BlockSpec((DH, DH), lambda i: (0, 0)),
            pl.BlockSpec((1, DH), lambda i: (0, 0)),
            pl.BlockSpec((DH, DH // 2), lambda i: (0, 0)),
            pl.BlockSpec((1, DH // 2), lambda i: (0, 0)),
            pl.BlockSpec((1, DH // 2), lambda i: (0, 0)),
            pl.BlockSpec((1, 1), lambda i: (0, 0)),
        ],
        out_specs=[
            pl.BlockSpec((RB, DH), lambda i: (i, 0)),
            pl.BlockSpec((1, RB), lambda i: (0, i)),
        ],
        out_shape=[
            jax.ShapeDtypeStruct((N, DH), jnp.float32),
            jax.ShapeDtypeStruct((1, N), jnp.float32),
        ],
    )(eh, nb, tw, L1, l1b, L2, l2b, A1, a1b, A2t, a2b)


def _s6_body(gl_ref, h_ref, lng_ref, lnb_ref, wc_ref, bc_ref,
             logits_ref, prob_ref, yhat_ref):
    gl = gl_ref[...]
    m = jnp.max(gl)
    e = jnp.exp(gl - m)
    alpha = e / jnp.sum(e)
    pooled = jnp.dot(alpha, h_ref[...],
                     preferred_element_type=jnp.float32)
    mu = jnp.mean(pooled, axis=1, keepdims=True)
    var = jnp.mean((pooled - mu) ** 2, axis=1, keepdims=True)
    normed = (pooled - mu) / jnp.sqrt(var + 1e-5) * lng_ref[...] + lnb_ref[...]
    logits = jnp.dot(normed, wc_ref[...],
                     preferred_element_type=jnp.float32) + bc_ref[...]
    logits_ref[...] = logits
    ml = jnp.max(logits, axis=1, keepdims=True)
    el = jnp.exp(logits - ml)
    prob_ref[...] = el / jnp.sum(el, axis=1, keepdims=True)
    iota2 = lax.broadcasted_iota(jnp.int32, (1, 2), 1)
    yhat_ref[...] = jnp.min(jnp.where(logits == ml, iota2, 2),
                            axis=1, keepdims=True)


def _stage6(gl, h, ln_g, ln_b, Wc, bc):
    return _PC(
        _s6_body,
        grid=(1,),
        in_specs=[
            pl.BlockSpec((1, N), lambda i: (0, 0)),
            pl.BlockSpec((N, DH), lambda i: (0, 0)),
            pl.BlockSpec((1, DH), lambda i: (0, 0)),
            pl.BlockSpec((1, DH), lambda i: (0, 0)),
            pl.BlockSpec((DH, 2), lambda i: (0, 0)),
            pl.BlockSpec((1, 2), lambda i: (0, 0)),
        ],
        out_specs=[
            pl.BlockSpec((1, 2), lambda i: (0, 0)),
            pl.BlockSpec((1, 2), lambda i: (0, 0)),
            pl.BlockSpec((1, 1), lambda i: (0, 0)),
        ],
        out_shape=[
            jax.ShapeDtypeStruct((1, 2), jnp.float32),
            jax.ShapeDtypeStruct((1, 2), jnp.float32),
            jax.ShapeDtypeStruct((1, 1), jnp.int32),
        ],
    )(gl, h, ln_g, ln_b, Wc, bc)


def kernel(x, W1, b1, Wh, bh, Wt, bt, L1, l1b, L2, l2b,
           A1, a1b, A2, a2b, ln_g, ln_b, Wc, bc):
    b1r = b1.reshape(1, DH)
    bhr = bh.reshape(1, DH)
    btr = bt.reshape(1, DH)
    l1br = l1b.reshape(1, DH)
    l2br = l2b.reshape(1, DH)
    a1br = a1b.reshape(1, DH // 2)
    a2t = A2.reshape(1, DH // 2)
    a2br = a2b.reshape(1, 1)
    lngr = ln_g.reshape(1, DH)
    lnbr = ln_b.reshape(1, DH)
    bcr = bc.reshape(1, 2)

    x1, csum = _stage1(x, W1, b1r)
    eh, et = _stage2(x1, csum, Wh, bhr, Wt, btr)
    tw_t, ti_t = _stage3(eh, et)
    tw = tw_t.T
    ti = ti_t.T
    nb = _sc_gather(et, ti.reshape(N * K))
    h, gl = _stage5(eh, nb, tw, L1, l1br, L2, l2br, A1, a1br, a2t, a2br)
    logits, prob, yhat = _stage6(gl, h, lngr, lnbr, Wc, bcr)
    return (logits, prob, yhat)


# speedup vs baseline: 1.1587x; 1.0215x over previous
import functools

import jax
import jax.numpy as jnp
from jax import lax
from jax.experimental import pallas as pl
from jax.experimental.pallas import tpu as pltpu
from jax.experimental.pallas import tpu_sc as plsc

N = 8192
DIN = 384
DH = 128
K = 8
RB = 256
CB = 1024
RB3 = 1024
NRB3 = N // RB3
NRB = N // RB
NCB = N // CB
SCALE = DH ** -0.5
NEG = float("-inf")

_PC = pl.pallas_call


def _lrelu(t):
    return jnp.where(t >= 0, t, 0.01 * t)


def _s1_body(x_ref, w_ref, b_ref, x1_ref, csum_ref):
    i = pl.program_id(0)
    z = jnp.dot(x_ref[...], w_ref[...], preferred_element_type=jnp.float32)
    x1 = _lrelu(z + b_ref[...])
    x1_ref[...] = x1

    @pl.when(i == 0)
    def _():
        csum_ref[...] = jnp.zeros_like(csum_ref)

    csum_ref[...] += jnp.sum(x1, axis=0, keepdims=True)


def _stage1(x, W1, b1):
    return _PC(
        _s1_body,
        grid=(NRB,),
        in_specs=[
            pl.BlockSpec((RB, DIN), lambda i: (i, 0)),
            pl.BlockSpec((DIN, DH), lambda i: (0, 0)),
            pl.BlockSpec((1, DH), lambda i: (0, 0)),
        ],
        out_specs=[
            pl.BlockSpec((RB, DH), lambda i: (i, 0)),
            pl.BlockSpec((1, DH), lambda i: (0, 0)),
        ],
        out_shape=[
            jax.ShapeDtypeStruct((N, DH), jnp.float32),
            jax.ShapeDtypeStruct((1, DH), jnp.float32),
        ],
    )(x, W1, b1)


def _s2_body(x1_ref, csum_ref, wh_ref, bh_ref, wt_ref, bt_ref, eh_ref, et_ref):
    x2 = (x1_ref[...] + csum_ref[...] * (1.0 / N)) * 0.5
    eh_ref[...] = jnp.dot(x2, wh_ref[...], preferred_element_type=jnp.float32) + bh_ref[...]
    et_ref[...] = jnp.dot(x2, wt_ref[...], preferred_element_type=jnp.float32) + bt_ref[...]


def _stage2(x1, csum, Wh, bh, Wt, bt):
    return _PC(
        _s2_body,
        grid=(NRB,),
        in_specs=[
            pl.BlockSpec((RB, DH), lambda i: (i, 0)),
            pl.BlockSpec((1, DH), lambda i: (0, 0)),
            pl.BlockSpec((DH, DH), lambda i: (0, 0)),
            pl.BlockSpec((1, DH), lambda i: (0, 0)),
            pl.BlockSpec((DH, DH), lambda i: (0, 0)),
            pl.BlockSpec((1, DH), lambda i: (0, 0)),
        ],
        out_specs=[
            pl.BlockSpec((RB, DH), lambda i: (i, 0)),
            pl.BlockSpec((RB, DH), lambda i: (i, 0)),
        ],
        out_shape=[
            jax.ShapeDtypeStruct((N, DH), jnp.float32),
            jax.ShapeDtypeStruct((N, DH), jnp.float32),
        ],
    )(x1, csum, Wh, bh, Wt, bt)


def _s3_body(eh_ref, et_ref, tw_ref, ti_ref):
    ehs = eh_ref[...] * SCALE
    iota_s = lax.broadcasted_iota(jnp.int32, (CB, RB3), 0)
    iota_k = lax.broadcasted_iota(jnp.int32, (K, RB3), 0)

    def col_step(c, carry):
        vals, ids = carry
        et_t = et_ref[pl.ds(c * CB, CB), :]
        logits = lax.dot_general(
            et_t, ehs, (((1,), (1,)), ((), ())),
            preferred_element_type=jnp.float32)

        def round_cond(st):
            r, work, m, vals, ids = st
            return jnp.logical_and(r < K, jnp.any(m > vals[K - 1:K, :]))

        def round_body(st):
            r, work, m, vals, ids = st
            cand = work == m
            pos = jnp.min(jnp.where(cand, iota_s, CB), axis=0, keepdims=True)
            work = jnp.where(iota_s == pos, NEG, work)
            cval = m
            cidx = pos + c * CB
            cnt = jnp.sum((vals >= cval).astype(jnp.int32), axis=0, keepdims=True)
            sv = jnp.concatenate(
                [jnp.full((1, RB3), NEG, jnp.float32), vals[: K - 1, :]], axis=0)
            si = jnp.concatenate(
                [jnp.zeros((1, RB3), jnp.int32), ids[: K - 1, :]], axis=0)
            newv = jnp.where(iota_k < cnt, vals,
                             jnp.where(iota_k == cnt, cval, sv))
            newi = jnp.where(iota_k < cnt, ids,
                             jnp.where(iota_k == cnt, cidx, si))
            m2 = jnp.max(work, axis=0, keepdims=True)
            return (r + 1, work, m2, newv, newi)

        m0 = jnp.max(logits, axis=0, keepdims=True)
        _, _, _, vals, ids = lax.while_loop(
            round_cond, round_body, (0, logits, m0, vals, ids))
        return (vals, ids)

    vals0 = jnp.full((K, RB3), NEG, jnp.float32)
    ids0 = jnp.zeros((K, RB3), jnp.int32)
    vals, ids = lax.fori_loop(0, NCB, col_step, (vals0, ids0))
    tw_ref[...] = vals
    ti_ref[...] = ids


def _stage3(eh, et):
    return _PC(
        _s3_body,
        grid=(NRB3,),
        in_specs=[
            pl.BlockSpec((RB3, DH), lambda i: (i, 0)),
            pl.BlockSpec((N, DH), lambda i: (0, 0)),
        ],
        out_specs=[
            pl.BlockSpec((K, RB3), lambda i: (0, i)),
            pl.BlockSpec((K, RB3), lambda i: (0, i)),
        ],
        out_shape=[
            jax.ShapeDtypeStruct((K, N), jnp.float32),
            jax.ShapeDtypeStruct((K, N), jnp.int32),
        ],
    )(eh, et)


_SC_CHUNK = 128


def _sc_gather(table, idx_flat):
    info = plsc.get_sparse_core_info()
    nw = info.num_cores * info.num_subcores
    b_per_w = (N * K) // nw
    n_chunks = b_per_w // _SC_CHUNK

    @functools.partial(
        pl.kernel,
        out_type=jax.ShapeDtypeStruct((N * K, DH), jnp.float32),
        mesh=plsc.VectorSubcoreMesh(core_axis_name="c", subcore_axis_name="s"),
        scratch_types=[
            pltpu.VMEM((_SC_CHUNK,), jnp.int32),
            pltpu.VMEM((_SC_CHUNK, DH), jnp.float32),
            pltpu.SemaphoreType.DMA,
        ],
    )
    def gather_kernel(table_hbm, idx_hbm, out_hbm, idx_v, rows_v, sem):
        wid = lax.axis_index("s") * info.num_cores + lax.axis_index("c")
        base = wid * b_per_w

        def body(ci, carry):
            off = base + ci * _SC_CHUNK
            pltpu.sync_copy(idx_hbm.at[pl.ds(off, _SC_CHUNK)], idx_v)
            pltpu.async_copy(table_hbm.at[idx_v], rows_v, sem).wait()
            pltpu.sync_copy(rows_v, out_hbm.at[pl.ds(off, _SC_CHUNK)])
            return carry

        lax.fori_loop(0, n_chunks, body, 0)

    return gather_kernel(table, idx_flat)


def _s5_body(eh_ref, nb_ref, tw_ref, l1_ref, l1b_ref, l2_ref, l2b_ref,
             a1_ref, a1b_ref, a2t_ref, a2b_ref, h_ref, gl_ref):
    eh = eh_ref[...]
    nb = nb_ref[...].reshape(RB, K, DH)
    tw = tw_ref[...]

    mw = jnp.max(tw, axis=1, keepdims=True)
    ew = jnp.exp(tw - mw)
    p = ew / jnp.sum(ew, axis=1, keepdims=True)

    eh3 = eh[:, None, :]
    ehr = p[:, :, None] * nb + (1.0 - p)[:, :, None] * eh3
    gate = jnp.tanh(eh3 + ehr)
    kw = jnp.sum(nb, axis=2) * jnp.sum(gate, axis=2)
    mk = jnp.max(kw, axis=1, keepdims=True)
    ek = jnp.exp(kw - mk)
    kp = ek / jnp.sum(ek, axis=1, keepdims=True)
    e_nh = jnp.sum(kp[:, :, None] * nb, axis=1)

    s_emb = _lrelu(jnp.dot(eh + e_nh, l1_ref[...],
                           preferred_element_type=jnp.float32) + l1b_ref[...])
    b_emb = _lrelu(jnp.dot(eh * e_nh, l2_ref[...],
                           preferred_element_type=jnp.float32) + l2b_ref[...])
    h = s_emb + b_emb
    h_ref[...] = h

    a = _lrelu(jnp.dot(h, a1_ref[...],
                       preferred_element_type=jnp.float32) + a1b_ref[...])
    gl = lax.dot_general(a2t_ref[...], a, (((1,), (1,)), ((), ())),
                         preferred_element_type=jnp.float32)
    gl_ref[...] = gl + a2b_ref[...]


def _stage5(eh, nb, tw, L1, l1b, L2, l2b, A1, a1b, A2t, a2b):
    return _PC(
        _s5_body,
        grid=(NRB,),
        in_specs=[
            pl.BlockSpec((RB, DH), lambda i: (i, 0)),
            pl.BlockSpec((RB * K, DH), lambda i: (i, 0)),
            pl.BlockSpec((RB, K), lambda i: (i, 0)),
            pl.BlockSpec((DH, DH), lambda i: (0, 0)),
            pl.BlockSpec((1, DH), lambda i: (0, 0)),
            pl.BlockSpec((DH, DH), lambda i: (0, 0)),
            pl.BlockSpec((1, DH), lambda i: (0, 0)),
            pl.BlockSpec((DH, DH // 2), lambda i: (0, 0)),
            pl.BlockSpec((1, DH // 2), lambda i: (0, 0)),
            pl.BlockSpec((1, DH // 2), lambda i: (0, 0)),
            pl.BlockSpec((1, 1), lambda i: (0, 0)),
        ],
        out_specs=[
            pl.BlockSpec((RB, DH), lambda i: (i, 0)),
            pl.BlockSpec((1, RB), lambda i: (0, i)),
        ],
        out_shape=[
            jax.ShapeDtypeStruct((N, DH), jnp.float32),
            jax.ShapeDtypeStruct((1, N), jnp.float32),
        ],
    )(eh, nb, tw, L1, l1b, L2, l2b, A1, a1b, A2t, a2b)


def _s6_body(gl_ref, h_ref, lng_ref, lnb_ref, wc_ref, bc_ref,
             logits_ref, prob_ref, yhat_ref):
    gl = gl_ref[...]
    m = jnp.max(gl)
    e = jnp.exp(gl - m)
    alpha = e / jnp.sum(e)
    pooled = jnp.dot(alpha, h_ref[...],
                     preferred_element_type=jnp.float32)
    mu = jnp.mean(pooled, axis=1, keepdims=True)
    var = jnp.mean((pooled - mu) ** 2, axis=1, keepdims=True)
    normed = (pooled - mu) / jnp.sqrt(var + 1e-5) * lng_ref[...] + lnb_ref[...]
    logits = jnp.dot(normed, wc_ref[...],
                     preferred_element_type=jnp.float32) + bc_ref[...]
    logits_ref[...] = logits
    ml = jnp.max(logits, axis=1, keepdims=True)
    el = jnp.exp(logits - ml)
    prob_ref[...] = el / jnp.sum(el, axis=1, keepdims=True)
    iota2 = lax.broadcasted_iota(jnp.int32, (1, 2), 1)
    yhat_ref[...] = jnp.min(jnp.where(logits == ml, iota2, 2),
                            axis=1, keepdims=True)


def _stage6(gl, h, ln_g, ln_b, Wc, bc):
    return _PC(
        _s6_body,
        grid=(1,),
        in_specs=[
            pl.BlockSpec((1, N), lambda i: (0, 0)),
            pl.BlockSpec((N, DH), lambda i: (0, 0)),
            pl.BlockSpec((1, DH), lambda i: (0, 0)),
            pl.BlockSpec((1, DH), lambda i: (0, 0)),
            pl.BlockSpec((DH, 2), lambda i: (0, 0)),
            pl.BlockSpec((1, 2), lambda i: (0, 0)),
        ],
        out_specs=[
            pl.BlockSpec((1, 2), lambda i: (0, 0)),
            pl.BlockSpec((1, 2), lambda i: (0, 0)),
            pl.BlockSpec((1, 1), lambda i: (0, 0)),
        ],
        out_shape=[
            jax.ShapeDtypeStruct((1, 2), jnp.float32),
            jax.ShapeDtypeStruct((1, 2), jnp.float32),
            jax.ShapeDtypeStruct((1, 1), jnp.int32),
        ],
    )(gl, h, ln_g, ln_b, Wc, bc)


def kernel(x, W1, b1, Wh, bh, Wt, bt, L1, l1b, L2, l2b,
           A1, a1b, A2, a2b, ln_g, ln_b, Wc, bc):
    b1r = b1.reshape(1, DH)
    bhr = bh.reshape(1, DH)
    btr = bt.reshape(1, DH)
    l1br = l1b.reshape(1, DH)
    l2br = l2b.reshape(1, DH)
    a1br = a1b.reshape(1, DH // 2)
    a2t = A2.reshape(1, DH // 2)
    a2br = a2b.reshape(1, 1)
    lngr = ln_g.reshape(1, DH)
    lnbr = ln_b.reshape(1, DH)
    bcr = bc.reshape(1, 2)

    x1, csum = _stage1(x, W1, b1r)
    eh, et = _stage2(x1, csum, Wh, bhr, Wt, btr)
    tw_t, ti_t = _stage3(eh, et)
    tw = tw_t.T
    ti = ti_t.T
    nb = _sc_gather(et, ti.reshape(N * K))
    h, gl = _stage5(eh, nb, tw, L1, l1br, L2, l2br, A1, a1br, a2t, a2br)
    logits, prob, yhat = _stage6(gl, h, lngr, lnbr, Wc, bcr)
    return (logits, prob, yhat)


# speedup vs baseline: 1.1601x; 1.0012x over previous
import functools

import jax
import jax.numpy as jnp
from jax import lax
from jax.experimental import pallas as pl
from jax.experimental.pallas import tpu as pltpu
from jax.experimental.pallas import tpu_sc as plsc

N = 8192
DIN = 384
DH = 128
K = 8
RB = 256
CB = 1024
RB3 = 1024
NRB3 = N // RB3
NRB = N // RB
NCB = N // CB
SCALE = DH ** -0.5
NEG = float("-inf")

_PC = pl.pallas_call


def _lrelu(t):
    return jnp.where(t >= 0, t, 0.01 * t)


def _s1_body(x_ref, w_ref, b_ref, x1_ref, csum_ref):
    i = pl.program_id(0)
    z = jnp.dot(x_ref[...], w_ref[...], preferred_element_type=jnp.float32)
    x1 = _lrelu(z + b_ref[...])
    x1_ref[...] = x1

    @pl.when(i == 0)
    def _():
        csum_ref[...] = jnp.zeros_like(csum_ref)

    csum_ref[...] += jnp.sum(x1, axis=0, keepdims=True)


def _stage1(x, W1, b1):
    return _PC(
        _s1_body,
        grid=(NRB,),
        in_specs=[
            pl.BlockSpec((RB, DIN), lambda i: (i, 0)),
            pl.BlockSpec((DIN, DH), lambda i: (0, 0)),
            pl.BlockSpec((1, DH), lambda i: (0, 0)),
        ],
        out_specs=[
            pl.BlockSpec((RB, DH), lambda i: (i, 0)),
            pl.BlockSpec((1, DH), lambda i: (0, 0)),
        ],
        out_shape=[
            jax.ShapeDtypeStruct((N, DH), jnp.float32),
            jax.ShapeDtypeStruct((1, DH), jnp.float32),
        ],
    )(x, W1, b1)


def _s2_body(x1_ref, csum_ref, wh_ref, bh_ref, wt_ref, bt_ref, eh_ref, et_ref):
    x2 = (x1_ref[...] + csum_ref[...] * (1.0 / N)) * 0.5
    eh_ref[...] = jnp.dot(x2, wh_ref[...], preferred_element_type=jnp.float32) + bh_ref[...]
    et_ref[...] = jnp.dot(x2, wt_ref[...], preferred_element_type=jnp.float32) + bt_ref[...]


def _stage2(x1, csum, Wh, bh, Wt, bt):
    return _PC(
        _s2_body,
        grid=(NRB,),
        in_specs=[
            pl.BlockSpec((RB, DH), lambda i: (i, 0)),
            pl.BlockSpec((1, DH), lambda i: (0, 0)),
            pl.BlockSpec((DH, DH), lambda i: (0, 0)),
            pl.BlockSpec((1, DH), lambda i: (0, 0)),
            pl.BlockSpec((DH, DH), lambda i: (0, 0)),
            pl.BlockSpec((1, DH), lambda i: (0, 0)),
        ],
        out_specs=[
            pl.BlockSpec((RB, DH), lambda i: (i, 0)),
            pl.BlockSpec((RB, DH), lambda i: (i, 0)),
        ],
        out_shape=[
            jax.ShapeDtypeStruct((N, DH), jnp.float32),
            jax.ShapeDtypeStruct((N, DH), jnp.float32),
        ],
    )(x1, csum, Wh, bh, Wt, bt)


def _s3_body(eh_ref, et_ref, tw_ref, ti_ref):
    ehs = eh_ref[...] * SCALE
    iota_s = lax.broadcasted_iota(jnp.int32, (CB, RB3), 0)
    iota_k = lax.broadcasted_iota(jnp.int32, (K, RB3), 0)

    def col_step(c, carry):
        vals, ids = carry
        et_t = et_ref[pl.ds(c * CB, CB), :]
        logits = lax.dot_general(
            et_t, ehs, (((1,), (1,)), ((), ())),
            preferred_element_type=jnp.float32)

        def round_cond(st):
            r, work, m, vals, ids = st
            return jnp.logical_and(r < K, jnp.any(m > vals[K - 1:K, :]))

        def round_body(st):
            r, work, m, vals, ids = st
            cand = work == m
            pos = jnp.min(jnp.where(cand, iota_s, CB), axis=0, keepdims=True)
            work = jnp.where(iota_s == pos, NEG, work)
            cval = m
            cidx = pos + c * CB
            cnt = jnp.sum((vals >= cval).astype(jnp.int32), axis=0, keepdims=True)
            sv = jnp.concatenate(
                [jnp.full((1, RB3), NEG, jnp.float32), vals[: K - 1, :]], axis=0)
            si = jnp.concatenate(
                [jnp.zeros((1, RB3), jnp.int32), ids[: K - 1, :]], axis=0)
            newv = jnp.where(iota_k < cnt, vals,
                             jnp.where(iota_k == cnt, cval, sv))
            newi = jnp.where(iota_k < cnt, ids,
                             jnp.where(iota_k == cnt, cidx, si))
            m2 = jnp.max(work, axis=0, keepdims=True)
            return (r + 1, work, m2, newv, newi)

        m0 = jnp.max(logits, axis=0, keepdims=True)
        _, _, _, vals, ids = lax.while_loop(
            round_cond, round_body, (0, logits, m0, vals, ids))
        return (vals, ids)

    vals0 = jnp.full((K, RB3), NEG, jnp.float32)
    ids0 = jnp.zeros((K, RB3), jnp.int32)
    vals, ids = lax.fori_loop(0, NCB, col_step, (vals0, ids0))
    tw_ref[...] = vals
    ti_ref[...] = ids


def _stage3(eh, et):
    return _PC(
        _s3_body,
        grid=(NRB3,),
        in_specs=[
            pl.BlockSpec((RB3, DH), lambda i: (i, 0)),
            pl.BlockSpec((N, DH), lambda i: (0, 0)),
        ],
        out_specs=[
            pl.BlockSpec((K, RB3), lambda i: (0, i)),
            pl.BlockSpec((K, RB3), lambda i: (0, i)),
        ],
        out_shape=[
            jax.ShapeDtypeStruct((K, N), jnp.float32),
            jax.ShapeDtypeStruct((K, N), jnp.int32),
        ],
    )(eh, et)


_SC_CHUNK = 128


def _sc_gather(table, idx2d):
    info = plsc.get_sparse_core_info()
    nw = info.num_cores * info.num_subcores
    rows_per_w = (N * K // _SC_CHUNK) // nw
    nbuf = 4

    @functools.partial(
        pl.kernel,
        out_type=jax.ShapeDtypeStruct((N * K, DH), jnp.float32),
        mesh=plsc.VectorSubcoreMesh(core_axis_name="c", subcore_axis_name="s"),
        scratch_types=[
            pltpu.VMEM((rows_per_w, _SC_CHUNK), jnp.int32),
            pltpu.VMEM((nbuf, _SC_CHUNK, DH), jnp.float32),
            pltpu.SemaphoreType.DMA,
            pltpu.SemaphoreType.DMA,
            pltpu.SemaphoreType.DMA,
            pltpu.SemaphoreType.DMA,
        ],
    )
    def gather_kernel(table_hbm, idx_hbm, out_hbm, idx_v, rows_v,
                      gs0, gs1, os0, os1):
        wid = lax.axis_index("s") * info.num_cores + lax.axis_index("c")
        rbase = wid * rows_per_w
        base = rbase * _SC_CHUNK
        pltpu.sync_copy(idx_hbm.at[pl.ds(rbase, rows_per_w)], idx_v)
        gs = [gs0, gs1]
        osm = [os0, os1]
        gh = {}
        oh = {}
        for ci in range(2):
            gh[ci] = pltpu.async_copy(
                table_hbm.at[idx_v.at[ci]], rows_v.at[ci % nbuf], gs[ci % 2])
        for ci in range(rows_per_w):
            gh[ci].wait()
            oh[ci] = pltpu.async_copy(
                rows_v.at[ci % nbuf],
                out_hbm.at[pl.ds(base + ci * _SC_CHUNK, _SC_CHUNK)],
                osm[ci % 2])
            if ci - 2 >= 0:
                oh[ci - 2].wait()
            if ci + 2 < rows_per_w:
                gh[ci + 2] = pltpu.async_copy(
                    table_hbm.at[idx_v.at[ci + 2]],
                    rows_v.at[(ci + 2) % nbuf], gs[ci % 2])
        oh[rows_per_w - 2].wait()
        oh[rows_per_w - 1].wait()

    return gather_kernel(table, idx2d)


def _s5_body(eh_ref, nb_ref, tw_ref, l1_ref, l1b_ref, l2_ref, l2b_ref,
             a1_ref, a1b_ref, a2t_ref, a2b_ref, h_ref, gl_ref):
    eh = eh_ref[...]
    nb = nb_ref[...].reshape(RB, K, DH)
    tw = tw_ref[...]

    mw = jnp.max(tw, axis=1, keepdims=True)
    ew = jnp.exp(tw - mw)
    p = ew / jnp.sum(ew, axis=1, keepdims=True)

    eh3 = eh[:, None, :]
    ehr = p[:, :, None] * nb + (1.0 - p)[:, :, None] * eh3
    gate = jnp.tanh(eh3 + ehr)
    kw = jnp.sum(nb, axis=2) * jnp.sum(gate, axis=2)
    mk = jnp.max(kw, axis=1, keepdims=True)
    ek = jnp.exp(kw - mk)
    kp = ek / jnp.sum(ek, axis=1, keepdims=True)
    e_nh = jnp.sum(kp[:, :, None] * nb, axis=1)

    s_emb = _lrelu(jnp.dot(eh + e_nh, l1_ref[...],
                           preferred_element_type=jnp.float32) + l1b_ref[...])
    b_emb = _lrelu(jnp.dot(eh * e_nh, l2_ref[...],
                           preferred_element_type=jnp.float32) + l2b_ref[...])
    h = s_emb + b_emb
    h_ref[...] = h

    a = _lrelu(jnp.dot(h, a1_ref[...],
                       preferred_element_type=jnp.float32) + a1b_ref[...])
    gl = lax.dot_general(a2t_ref[...], a, (((1,), (1,)), ((), ())),
                         preferred_element_type=jnp.float32)
    gl_ref[...] = gl + a2b_ref[...]


def _stage5(eh, nb, tw, L1, l1b, L2, l2b, A1, a1b, A2t, a2b):
    return _PC(
        _s5_body,
        grid=(NRB,),
        in_specs=[
            pl.BlockSpec((RB, DH), lambda i: (i, 0)),
            pl.BlockSpec((RB * K, DH), lambda i: (i, 0)),
            pl.BlockSpec((RB, K), lambda i: (i, 0)),
            pl.BlockSpec((DH, DH), lambda i: (0, 0)),
            pl.BlockSpec((1, DH), lambda i: (0, 0)),
            pl.BlockSpec((DH, DH), lambda i: (0, 0)),
            pl.BlockSpec((1, DH), lambda i: (0, 0)),
            pl.BlockSpec((DH, DH // 2), lambda i: (0, 0)),
            pl.BlockSpec((1, DH // 2), lambda i: (0, 0)),
            pl.BlockSpec((1, DH // 2), lambda i: (0, 0)),
            pl.BlockSpec((1, 1), lambda i: (0, 0)),
        ],
        out_specs=[
            pl.BlockSpec((RB, DH), lambda i: (i, 0)),
            pl.BlockSpec((1, RB), lambda i: (0, i)),
        ],
        out_shape=[
            jax.ShapeDtypeStruct((N, DH), jnp.float32),
            jax.ShapeDtypeStruct((1, N), jnp.float32),
        ],
    )(eh, nb, tw, L1, l1b, L2, l2b, A1, a1b, A2t, a2b)


def _s6_body(gl_ref, h_ref, lng_ref, lnb_ref, wc_ref, bc_ref,
             logits_ref, prob_ref, yhat_ref):
    gl = gl_ref[...]
    m = jnp.max(gl)
    e = jnp.exp(gl - m)
    alpha = e / jnp.sum(e)
    pooled = jnp.dot(alpha, h_ref[...],
                     preferred_element_type=jnp.float32)
    mu = jnp.mean(pooled, axis=1, keepdims=True)
    var = jnp.mean((pooled - mu) ** 2, axis=1, keepdims=True)
    normed = (pooled - mu) / jnp.sqrt(var + 1e-5) * lng_ref[...] + lnb_ref[...]
    logits = jnp.dot(normed, wc_ref[...],
                     preferred_element_type=jnp.float32) + bc_ref[...]
    logits_ref[...] = logits
    ml = jnp.max(logits, axis=1, keepdims=True)
    el = jnp.exp(logits - ml)
    prob_ref[...] = el / jnp.sum(el, axis=1, keepdims=True)
    iota2 = lax.broadcasted_iota(jnp.int32, (1, 2), 1)
    yhat_ref[...] = jnp.min(jnp.where(logits == ml, iota2, 2),
                            axis=1, keepdims=True)


def _stage6(gl, h, ln_g, ln_b, Wc, bc):
    return _PC(
        _s6_body,
        grid=(1,),
        in_specs=[
            pl.BlockSpec((1, N), lambda i: (0, 0)),
            pl.BlockSpec((N, DH), lambda i: (0, 0)),
            pl.BlockSpec((1, DH), lambda i: (0, 0)),
            pl.BlockSpec((1, DH), lambda i: (0, 0)),
            pl.BlockSpec((DH, 2), lambda i: (0, 0)),
            pl.BlockSpec((1, 2), lambda i: (0, 0)),
        ],
        out_specs=[
            pl.BlockSpec((1, 2), lambda i: (0, 0)),
            pl.BlockSpec((1, 2), lambda i: (0, 0)),
            pl.BlockSpec((1, 1), lambda i: (0, 0)),
        ],
        out_shape=[
            jax.ShapeDtypeStruct((1, 2), jnp.float32),
            jax.ShapeDtypeStruct((1, 2), jnp.float32),
            jax.ShapeDtypeStruct((1, 1), jnp.int32),
        ],
    )(gl, h, ln_g, ln_b, Wc, bc)


def kernel(x, W1, b1, Wh, bh, Wt, bt, L1, l1b, L2, l2b,
           A1, a1b, A2, a2b, ln_g, ln_b, Wc, bc):
    b1r = b1.reshape(1, DH)
    bhr = bh.reshape(1, DH)
    btr = bt.reshape(1, DH)
    l1br = l1b.reshape(1, DH)
    l2br = l2b.reshape(1, DH)
    a1br = a1b.reshape(1, DH // 2)
    a2t = A2.reshape(1, DH // 2)
    a2br = a2b.reshape(1, 1)
    lngr = ln_g.reshape(1, DH)
    lnbr = ln_b.reshape(1, DH)
    bcr = bc.reshape(1, 2)

    x1, csum = _stage1(x, W1, b1r)
    eh, et = _stage2(x1, csum, Wh, bhr, Wt, btr)
    tw_t, ti_t = _stage3(eh, et)
    tw = tw_t.T
    ti = ti_t.T
    nb = _sc_gather(et, ti.reshape(N * K // _SC_CHUNK, _SC_CHUNK))
    h, gl = _stage5(eh, nb, tw, L1, l1br, L2, l2br, A1, a1br, a2t, a2br)
    logits, prob, yhat = _stage6(gl, h, lngr, lnbr, Wc, bcr)
    return (logits, prob, yhat)


# speedup vs baseline: 1.2328x; 1.0626x over previous
import functools

import jax
import jax.numpy as jnp
from jax import lax
from jax.experimental import pallas as pl
from jax.experimental.pallas import tpu as pltpu
from jax.experimental.pallas import tpu_sc as plsc

N = 8192
DIN = 384
DH = 128
K = 8
RB = 256
CB = 1024
RB3 = 1024
NRB3 = N // RB3
NRB = N // RB
NCB = N // CB
SCALE = DH ** -0.5
NEG = float("-inf")

_PC = pl.pallas_call


def _lrelu(t):
    return jnp.where(t >= 0, t, 0.01 * t)


def _s1_body(x_ref, w_ref, b_ref, wh_ref, wt_ref, a_ref, b2_ref, csum_ref):
    i = pl.program_id(0)
    z = jnp.dot(x_ref[...], w_ref[...], preferred_element_type=jnp.float32)
    x1 = _lrelu(z + b_ref[...])
    a_ref[...] = jnp.dot(x1, wh_ref[...], preferred_element_type=jnp.float32)
    b2_ref[...] = jnp.dot(x1, wt_ref[...], preferred_element_type=jnp.float32)

    @pl.when(i == 0)
    def _():
        csum_ref[...] = jnp.zeros_like(csum_ref)

    csum_ref[...] += jnp.sum(x1, axis=0, keepdims=True)


def _stage1(x, W1, b1, Wh, Wt):
    return _PC(
        _s1_body,
        grid=(NRB,),
        in_specs=[
            pl.BlockSpec((RB, DIN), lambda i: (i, 0)),
            pl.BlockSpec((DIN, DH), lambda i: (0, 0)),
            pl.BlockSpec((1, DH), lambda i: (0, 0)),
            pl.BlockSpec((DH, DH), lambda i: (0, 0)),
            pl.BlockSpec((DH, DH), lambda i: (0, 0)),
        ],
        out_specs=[
            pl.BlockSpec((RB, DH), lambda i: (i, 0)),
            pl.BlockSpec((RB, DH), lambda i: (i, 0)),
            pl.BlockSpec((1, DH), lambda i: (0, 0)),
        ],
        out_shape=[
            jax.ShapeDtypeStruct((N, DH), jnp.float32),
            jax.ShapeDtypeStruct((N, DH), jnp.float32),
            jax.ShapeDtypeStruct((1, DH), jnp.float32),
        ],
    )(x, W1, b1, Wh, Wt)


def _s2_body(a_ref, b_ref, csum_ref, wh_ref, bh_ref, wt_ref, bt_ref,
             eh_ref, et_ref):
    mean = csum_ref[...] * (0.5 / N)
    rh = jnp.dot(mean, wh_ref[...], preferred_element_type=jnp.float32) + bh_ref[...]
    rt = jnp.dot(mean, wt_ref[...], preferred_element_type=jnp.float32) + bt_ref[...]
    eh_ref[...] = 0.5 * a_ref[...] + rh
    et_ref[...] = 0.5 * b_ref[...] + rt


def _stage2(a, b, csum, Wh, bh, Wt, bt):
    return _PC(
        _s2_body,
        grid=(NRB,),
        in_specs=[
            pl.BlockSpec((RB, DH), lambda i: (i, 0)),
            pl.BlockSpec((RB, DH), lambda i: (i, 0)),
            pl.BlockSpec((1, DH), lambda i: (0, 0)),
            pl.BlockSpec((DH, DH), lambda i: (0, 0)),
            pl.BlockSpec((1, DH), lambda i: (0, 0)),
            pl.BlockSpec((DH, DH), lambda i: (0, 0)),
            pl.BlockSpec((1, DH), lambda i: (0, 0)),
        ],
        out_specs=[
            pl.BlockSpec((RB, DH), lambda i: (i, 0)),
            pl.BlockSpec((RB, DH), lambda i: (i, 0)),
        ],
        out_shape=[
            jax.ShapeDtypeStruct((N, DH), jnp.float32),
            jax.ShapeDtypeStruct((N, DH), jnp.float32),
        ],
    )(a, b, csum, Wh, bh, Wt, bt)


def _s3_body(eh_ref, et_ref, tw_ref, ti_ref):
    ehs = eh_ref[...] * SCALE
    iota_s = lax.broadcasted_iota(jnp.int32, (CB, RB3), 0)
    iota_k = lax.broadcasted_iota(jnp.int32, (K, RB3), 0)

    def col_step(c, carry):
        vals, ids = carry
        et_t = et_ref[pl.ds(c * CB, CB), :]
        logits = lax.dot_general(
            et_t, ehs, (((1,), (1,)), ((), ())),
            preferred_element_type=jnp.float32)

        def round_cond(st):
            r, work, m, vals, ids = st
            return jnp.logical_and(r < K, jnp.any(m > vals[K - 1:K, :]))

        def round_body(st):
            r, work, m, vals, ids = st
            cand = work == m
            pos = jnp.min(jnp.where(cand, iota_s, CB), axis=0, keepdims=True)
            work = jnp.where(iota_s == pos, NEG, work)
            cval = m
            cidx = pos + c * CB
            cnt = jnp.sum((vals >= cval).astype(jnp.int32), axis=0, keepdims=True)
            sv = jnp.concatenate(
                [jnp.full((1, RB3), NEG, jnp.float32), vals[: K - 1, :]], axis=0)
            si = jnp.concatenate(
                [jnp.zeros((1, RB3), jnp.int32), ids[: K - 1, :]], axis=0)
            newv = jnp.where(iota_k < cnt, vals,
                             jnp.where(iota_k == cnt, cval, sv))
            newi = jnp.where(iota_k < cnt, ids,
                             jnp.where(iota_k == cnt, cidx, si))
            m2 = jnp.max(work, axis=0, keepdims=True)
            return (r + 1, work, m2, newv, newi)

        m0 = jnp.max(logits, axis=0, keepdims=True)
        _, _, _, vals, ids = lax.while_loop(
            round_cond, round_body, (0, logits, m0, vals, ids))
        return (vals, ids)

    vals0 = jnp.full((K, RB3), NEG, jnp.float32)
    ids0 = jnp.zeros((K, RB3), jnp.int32)
    vals, ids = lax.fori_loop(0, NCB, col_step, (vals0, ids0))
    tw_ref[...] = vals
    ti_ref[...] = ids


def _stage3(eh, et):
    return _PC(
        _s3_body,
        grid=(NRB3,),
        in_specs=[
            pl.BlockSpec((RB3, DH), lambda i: (i, 0)),
            pl.BlockSpec((N, DH), lambda i: (0, 0)),
        ],
        out_specs=[
            pl.BlockSpec((K, RB3), lambda i: (0, i)),
            pl.BlockSpec((K, RB3), lambda i: (0, i)),
        ],
        out_shape=[
            jax.ShapeDtypeStruct((K, N), jnp.float32),
            jax.ShapeDtypeStruct((K, N), jnp.int32),
        ],
    )(eh, et)


_SC_CHUNK = 128


def _sc_gather(table, idx2d):
    info = plsc.get_sparse_core_info()
    nw = info.num_cores * info.num_subcores
    rows_per_w = (N * K // _SC_CHUNK) // nw
    nbuf = 4

    @functools.partial(
        pl.kernel,
        out_type=jax.ShapeDtypeStruct((N * K, DH), jnp.float32),
        mesh=plsc.VectorSubcoreMesh(core_axis_name="c", subcore_axis_name="s"),
        scratch_types=[
            pltpu.VMEM((rows_per_w, _SC_CHUNK), jnp.int32),
            pltpu.VMEM((nbuf, _SC_CHUNK, DH), jnp.float32),
            pltpu.SemaphoreType.DMA,
            pltpu.SemaphoreType.DMA,
            pltpu.SemaphoreType.DMA,
            pltpu.SemaphoreType.DMA,
        ],
    )
    def gather_kernel(table_hbm, idx_hbm, out_hbm, idx_v, rows_v,
                      gs0, gs1, os0, os1):
        wid = lax.axis_index("s") * info.num_cores + lax.axis_index("c")
        rbase = wid * rows_per_w
        base = rbase * _SC_CHUNK
        pltpu.sync_copy(idx_hbm.at[pl.ds(rbase, rows_per_w)], idx_v)
        gs = [gs0, gs1]
        osm = [os0, os1]
        gh = {}
        oh = {}
        for ci in range(2):
            gh[ci] = pltpu.async_copy(
                table_hbm.at[idx_v.at[ci]], rows_v.at[ci % nbuf], gs[ci % 2])
        for ci in range(rows_per_w):
            gh[ci].wait()
            oh[ci] = pltpu.async_copy(
                rows_v.at[ci % nbuf],
                out_hbm.at[pl.ds(base + ci * _SC_CHUNK, _SC_CHUNK)],
                osm[ci % 2])
            if ci - 2 >= 0:
                oh[ci - 2].wait()
            if ci + 2 < rows_per_w:
                gh[ci + 2] = pltpu.async_copy(
                    table_hbm.at[idx_v.at[ci + 2]],
                    rows_v.at[(ci + 2) % nbuf], gs[ci % 2])
        oh[rows_per_w - 2].wait()
        oh[rows_per_w - 1].wait()

    return gather_kernel(table, idx2d)


def _s5_body(eh_ref, nb_ref, tw_ref, l1_ref, l1b_ref, l2_ref, l2b_ref,
             a1_ref, a1b_ref, a2t_ref, a2b_ref, h_ref, gl_ref):
    eh = eh_ref[...]
    nb = nb_ref[...].reshape(RB, K, DH)
    tw = tw_ref[...]

    mw = jnp.max(tw, axis=1, keepdims=True)
    ew = jnp.exp(tw - mw)
    p = ew / jnp.sum(ew, axis=1, keepdims=True)

    eh3 = eh[:, None, :]
    ehr = p[:, :, None] * nb + (1.0 - p)[:, :, None] * eh3
    gate = jnp.tanh(eh3 + ehr)
    kw = jnp.sum(nb, axis=2) * jnp.sum(gate, axis=2)
    mk = jnp.max(kw, axis=1, keepdims=True)
    ek = jnp.exp(kw - mk)
    kp = ek / jnp.sum(ek, axis=1, keepdims=True)
    e_nh = jnp.sum(kp[:, :, None] * nb, axis=1)

    s_emb = _lrelu(jnp.dot(eh + e_nh, l1_ref[...],
                           preferred_element_type=jnp.float32) + l1b_ref[...])
    b_emb = _lrelu(jnp.dot(eh * e_nh, l2_ref[...],
                           preferred_element_type=jnp.float32) + l2b_ref[...])
    h = s_emb + b_emb
    h_ref[...] = h

    a = _lrelu(jnp.dot(h, a1_ref[...],
                       preferred_element_type=jnp.float32) + a1b_ref[...])
    gl = lax.dot_general(a2t_ref[...], a, (((1,), (1,)), ((), ())),
                         preferred_element_type=jnp.float32)
    gl_ref[...] = gl + a2b_ref[...]


def _stage5(eh, nb, tw, L1, l1b, L2, l2b, A1, a1b, A2t, a2b):
    return _PC(
        _s5_body,
        grid=(NRB,),
        in_specs=[
            pl.BlockSpec((RB, DH), lambda i: (i, 0)),
            pl.BlockSpec((RB * K, DH), lambda i: (i, 0)),
            pl.BlockSpec((RB, K), lambda i: (i, 0)),
            pl.BlockSpec((DH, DH), lambda i: (0, 0)),
            pl.BlockSpec((1, DH), lambda i: (0, 0)),
            pl.BlockSpec((DH, DH), lambda i: (0, 0)),
            pl.BlockSpec((1, DH), lambda i: (0, 0)),
            pl.BlockSpec((DH, DH // 2), lambda i: (0, 0)),
            pl.BlockSpec((1, DH // 2), lambda i: (0, 0)),
            pl.BlockSpec((1, DH // 2), lambda i: (0, 0)),
            pl.BlockSpec((1, 1), lambda i: (0, 0)),
        ],
        out_specs=[
            pl.BlockSpec((RB, DH), lambda i: (i, 0)),
            pl.BlockSpec((1, RB), lambda i: (0, i)),
        ],
        out_shape=[
            jax.ShapeDtypeStruct((N, DH), jnp.float32),
            jax.ShapeDtypeStruct((1, N), jnp.float32),
        ],
    )(eh, nb, tw, L1, l1b, L2, l2b, A1, a1b, A2t, a2b)


def _s6_body(gl_ref, h_ref, lng_ref, lnb_ref, wc_ref, bc_ref,
             logits_ref, prob_ref, yhat_ref):
    gl = gl_ref[...]
    m = jnp.max(gl)
    e = jnp.exp(gl - m)
    alpha = e / jnp.sum(e)
    pooled = jnp.dot(alpha, h_ref[...],
                     preferred_element_type=jnp.float32)
    mu = jnp.mean(pooled, axis=1, keepdims=True)
    var = jnp.mean((pooled - mu) ** 2, axis=1, keepdims=True)
    normed = (pooled - mu) / jnp.sqrt(var + 1e-5) * lng_ref[...] + lnb_ref[...]
    logits = jnp.dot(normed, wc_ref[...],
                     preferred_element_type=jnp.float32) + bc_ref[...]
    logits_ref[...] = logits
    ml = jnp.max(logits, axis=1, keepdims=True)
    el = jnp.exp(logits - ml)
    prob_ref[...] = el / jnp.sum(el, axis=1, keepdims=True)
    iota2 = lax.broadcasted_iota(jnp.int32, (1, 2), 1)
    yhat_ref[...] = jnp.min(jnp.where(logits == ml, iota2, 2),
                            axis=1, keepdims=True)


def _stage6(gl, h, ln_g, ln_b, Wc, bc):
    return _PC(
        _s6_body,
        grid=(1,),
        in_specs=[
            pl.BlockSpec((1, N), lambda i: (0, 0)),
            pl.BlockSpec((N, DH), lambda i: (0, 0)),
            pl.BlockSpec((1, DH), lambda i: (0, 0)),
            pl.BlockSpec((1, DH), lambda i: (0, 0)),
            pl.BlockSpec((DH, 2), lambda i: (0, 0)),
            pl.BlockSpec((1, 2), lambda i: (0, 0)),
        ],
        out_specs=[
            pl.BlockSpec((1, 2), lambda i: (0, 0)),
            pl.BlockSpec((1, 2), lambda i: (0, 0)),
            pl.BlockSpec((1, 1), lambda i: (0, 0)),
        ],
        out_shape=[
            jax.ShapeDtypeStruct((1, 2), jnp.float32),
            jax.ShapeDtypeStruct((1, 2), jnp.float32),
            jax.ShapeDtypeStruct((1, 1), jnp.int32),
        ],
    )(gl, h, ln_g, ln_b, Wc, bc)


def kernel(x, W1, b1, Wh, bh, Wt, bt, L1, l1b, L2, l2b,
           A1, a1b, A2, a2b, ln_g, ln_b, Wc, bc):
    b1r = b1.reshape(1, DH)
    bhr = bh.reshape(1, DH)
    btr = bt.reshape(1, DH)
    l1br = l1b.reshape(1, DH)
    l2br = l2b.reshape(1, DH)
    a1br = a1b.reshape(1, DH // 2)
    a2t = A2.reshape(1, DH // 2)
    a2br = a2b.reshape(1, 1)
    lngr = ln_g.reshape(1, DH)
    lnbr = ln_b.reshape(1, DH)
    bcr = bc.reshape(1, 2)

    a, b, csum = _stage1(x, W1, b1r, Wh, Wt)
    eh, et = _stage2(a, b, csum, Wh, bhr, Wt, btr)
    tw_t, ti_t = _stage3(eh, et)
    tw = tw_t.T
    ti = ti_t.T
    nb = _sc_gather(et, ti.reshape(N * K // _SC_CHUNK, _SC_CHUNK))
    h, gl = _stage5(eh, nb, tw, L1, l1br, L2, l2br, A1, a1br, a2t, a2br)
    logits, prob, yhat = _stage6(gl, h, lngr, lnbr, Wc, bcr)
    return (logits, prob, yhat)
